# baseline (speedup 1.0000x reference)
import jax
import jax.numpy as jnp
from jax import lax
from jax.experimental import pallas as pl
from jax.experimental.pallas import tpu as pltpu
from jax.experimental.pallas import tpu_sc as plsc

N_NODES = 10000
N_EDGES = 320000
FDIM = 128

NC = 2
NS = 16
NW = NC * NS
K = 128
NPAD = 10240
SLAB = NPAD // NS
CHUNKS = 80
HALF = CHUNKS // 2
PAIRS = HALF // 2 - 1
EPW = CHUNKS * K
E_PAD = EPW * NW
EROWS = NW * CHUNKS

R = 256
GRID = NPAD // R


def _sc_agg_body(with_deg, table, ei, z2, z1, agg_out, deg_out,
                 s_idx, d_idx, rows0, rows1, ones, acc, deg_acc, sem0, sem1):
    c = lax.axis_index("c")
    s = lax.axis_index("s")
    slab = s * SLAB
    w = c * NS + s
    pltpu.sync_copy(z2.at[pl.ds(slab, SLAB)], acc.at[pl.ds(slab, SLAB)])
    if with_deg:
        pltpu.sync_copy(z1.at[pl.ds(slab, SLAB)], deg_acc.at[pl.ds(slab, SLAB)])
        for i in range(K // 16):
            ones[pl.ds(i * 16, 16)] = jnp.full((16,), 1.0, jnp.float32)
    plsc.subcore_barrier()

    def gather_start(j, rows, sem):
        pltpu.async_copy(table.at[s_idx.at[j]], rows, sem)

    def gather_wait(j, rows, sem):
        pltpu.make_async_copy(table.at[s_idx.at[j]], rows, sem).wait()

    def scatter(j, rows):
        pltpu.sync_copy(rows, acc.at[d_idx.at[j]], add=True)
        if with_deg:
            pltpu.sync_copy(ones, deg_acc.at[d_idx.at[j]], add=True)

    for phase in range(CHUNKS // HALF):
        row0 = w * CHUNKS + phase * HALF
        pltpu.sync_copy(ei.at[pl.ds(row0, HALF)], s_idx)
        pltpu.sync_copy(ei.at[pl.ds(EROWS + row0, HALF)], d_idx)

        gather_start(0, rows0, sem0)
        gather_start(1, rows1, sem1)

        def pair(t, carry):
            j = 2 * t
            gather_wait(j, rows0, sem0)
            scatter(j, rows0)
            gather_start(j + 2, rows0, sem0)
            gather_wait(j + 1, rows1, sem1)
            scatter(j + 1, rows1)
            gather_start(j + 3, rows1, sem1)
            return carry

        lax.fori_loop(0, PAIRS, pair, 0)
        gather_wait(HALF - 2, rows0, sem0)
        scatter(HALF - 2, rows0)
        gather_wait(HALF - 1, rows1, sem1)
        scatter(HALF - 1, rows1)

    plsc.subcore_barrier()
    out_off = c * NPAD + slab
    pltpu.sync_copy(acc.at[pl.ds(slab, SLAB)], agg_out.at[pl.ds(out_off, SLAB)])
    if with_deg:
        pltpu.sync_copy(deg_acc.at[pl.ds(slab, SLAB)],
                        deg_out.at[pl.ds(out_off, SLAB)])


def _sc_aggregate(table, ei2d, z2, z1, with_deg):
    mesh = plsc.VectorSubcoreMesh(core_axis_name="c", subcore_axis_name="s")
    out_type = [jax.ShapeDtypeStruct((NC * NPAD, FDIM), jnp.float32)]
    scratch = [
        pltpu.VMEM((HALF, K), jnp.int32),
        pltpu.VMEM((HALF, K), jnp.int32),
        pltpu.VMEM((K, FDIM), jnp.float32),
        pltpu.VMEM((K, FDIM), jnp.float32),
        pltpu.VMEM((K,), jnp.float32),
        pltpu.VMEM_SHARED((NPAD, FDIM), jnp.float32),
        pltpu.VMEM_SHARED((NPAD,), jnp.float32),
        pltpu.SemaphoreType.DMA,
        pltpu.SemaphoreType.DMA,
    ]
    if with_deg:
        out_type.append(jax.ShapeDtypeStruct((NC * NPAD,), jnp.float32))

        def body(table, ei, z2, z1, agg_out, deg_out, *rest):
            _sc_agg_body(True, table, ei, z2, z1, agg_out, deg_out, *rest)
    else:
        def body(table, ei, z2, z1, agg_out, *rest):
            _sc_agg_body(False, table, ei, z2, z1, agg_out, None, *rest)

    f = pl.kernel(body, out_type=tuple(out_type), mesh=mesh,
                  scratch_types=scratch)
    return f(table, ei2d, z2, z1)


def _mm_relu_body(a0, a1, d0, d1, w, b, o):
    d = jnp.maximum(d0[...] + d1[...], 1.0)
    h = (a0[...] + a1[...]) / d
    y = jnp.dot(h, w[...], preferred_element_type=jnp.float32) + b[...]
    o[...] = jnp.maximum(y, 0.0)


def _mm_fused_body(a0, a1, d0, d1, w2, b2, w3, b3, o):
    d = jnp.maximum(d0[...] + d1[...], 1.0)
    h = (a0[...] + a1[...]) / d
    h = jnp.maximum(jnp.dot(h, w2[...], preferred_element_type=jnp.float32) + b2[...], 0.0)
    o[...] = jnp.dot(h, w3[...], preferred_element_type=jnp.float32) + b3[...]


_A0 = pl.BlockSpec((R, FDIM), lambda i: (i, 0))
_A1 = pl.BlockSpec((R, FDIM), lambda i: (i + GRID, 0))
_D0 = pl.BlockSpec((R, 1), lambda i: (i, 0))
_D1 = pl.BlockSpec((R, 1), lambda i: (i + GRID, 0))
_W = pl.BlockSpec((FDIM, FDIM), lambda i: (0, 0))
_B = pl.BlockSpec((1, FDIM), lambda i: (0, 0))
_O = pl.BlockSpec((R, FDIM), lambda i: (i, 0))


def _mm_relu(agg, deg2, w, b):
    return pl.pallas_call(
        _mm_relu_body,
        grid=(GRID,),
        in_specs=[_A0, _A1, _D0, _D1, _W, _B],
        out_specs=_O,
        out_shape=jax.ShapeDtypeStruct((NPAD, FDIM), jnp.float32),
    )(agg, agg, deg2, deg2, w, b)


def _mm_fused(agg, deg2, w2, b2, w3, b3):
    return pl.pallas_call(
        _mm_fused_body,
        grid=(GRID,),
        in_specs=[_A0, _A1, _D0, _D1, _W, _B, _W, _B],
        out_specs=_O,
        out_shape=jax.ShapeDtypeStruct((NPAD, FDIM), jnp.float32),
    )(agg, agg, deg2, deg2, w2, b2, w3, b3)


def kernel(x, edge_index, W1, b1, W2, b2, W3, b3):
    src = jnp.pad(edge_index[0].astype(jnp.int32), (0, E_PAD - N_EDGES),
                  constant_values=0)
    dst = jnp.pad(edge_index[1].astype(jnp.int32), (0, E_PAD - N_EDGES),
                  constant_values=N_NODES)
    ei2d = jnp.concatenate([src, dst]).reshape(2 * EROWS, K)
    z2 = jnp.zeros((NPAD, FDIM), jnp.float32)
    z1 = jnp.zeros((NPAD,), jnp.float32)

    agg1, deg = _sc_aggregate(x, ei2d, z2, z1, with_deg=True)
    deg2 = deg.reshape(NC * NPAD, 1)
    h1 = _mm_relu(agg1, deg2, W1, b1[None, :])
    (agg2,) = _sc_aggregate(h1, ei2d, z2, z1, with_deg=False)
    out = _mm_fused(agg2, deg2, W2, b2[None, :], W3, b3[None, :])
    return out[:N_NODES]


# speedup vs baseline: 2.9745x; 2.9745x over previous
import jax
import jax.numpy as jnp
from jax import lax
from jax.experimental import pallas as pl
from jax.experimental.pallas import tpu as pltpu
from jax.experimental.pallas import tpu_sc as plsc

N_NODES = 10000
N_EDGES = 320000
FDIM = 128

NC = 2
NS = 16
NW = NC * NS
K = 128
NPAD = 10240
SLAB = NPAD // NS
CHUNKS = 80
HALF = CHUNKS // 2
PAIRS = HALF // 2 - 1
EPW = CHUNKS * K
E_PAD = EPW * NW
EROWS = NW * CHUNKS

R = 256
GRID = NPAD // R


def _sc_agg_body(with_deg, table, ei, z2, z1, agg_out, deg_out,
                 s_idx, d_idx, rows0, rows1, ones, acc, deg_acc, sem0, sem1):
    c = lax.axis_index("c")
    s = lax.axis_index("s")
    slab = s * SLAB
    w = c * NS + s
    pltpu.sync_copy(z2.at[pl.ds(slab, SLAB)], acc.at[pl.ds(slab, SLAB)])
    if with_deg:
        pltpu.sync_copy(z1.at[pl.ds(slab, SLAB)], deg_acc.at[pl.ds(slab, SLAB)])
        for i in range(K // 16):
            ones[pl.ds(i * 16, 16)] = jnp.full((16,), 1.0, jnp.float32)
    plsc.subcore_barrier()

    def gather_start(j, rows, sem):
        pltpu.async_copy(table.at[s_idx.at[j]], rows, sem)

    def gather_wait(j, rows, sem):
        pltpu.make_async_copy(table.at[s_idx.at[j]], rows, sem).wait()

    def scatter(j, rows):
        pltpu.sync_copy(rows, acc.at[d_idx.at[j]], add=True)
        if with_deg:
            pltpu.sync_copy(ones, deg_acc.at[d_idx.at[j]], add=True)

    for phase in range(CHUNKS // HALF):
        row0 = w * CHUNKS + phase * HALF
        pltpu.sync_copy(ei.at[pl.ds(row0, HALF)], s_idx)
        pltpu.sync_copy(ei.at[pl.ds(EROWS + row0, HALF)], d_idx)

        gather_start(0, rows0, sem0)
        gather_start(1, rows1, sem1)

        def pair(t, carry):
            j = 2 * t
            gather_wait(j, rows0, sem0)
            scatter(j, rows0)
            gather_start(j + 2, rows0, sem0)
            gather_wait(j + 1, rows1, sem1)
            scatter(j + 1, rows1)
            gather_start(j + 3, rows1, sem1)
            return carry

        lax.fori_loop(0, PAIRS, pair, 0)
        gather_wait(HALF - 2, rows0, sem0)
        scatter(HALF - 2, rows0)
        gather_wait(HALF - 1, rows1, sem1)
        scatter(HALF - 1, rows1)

    plsc.subcore_barrier()
    out_off = c * NPAD + slab
    pltpu.sync_copy(acc.at[pl.ds(slab, SLAB)], agg_out.at[pl.ds(out_off, SLAB)])
    if with_deg:
        pltpu.sync_copy(deg_acc.at[pl.ds(slab, SLAB)],
                        deg_out.at[pl.ds(out_off, SLAB)])


def _sc_aggregate(table, ei2d, z2, z1, with_deg):
    mesh = plsc.VectorSubcoreMesh(core_axis_name="c", subcore_axis_name="s")
    out_type = [jax.ShapeDtypeStruct((NC * NPAD, FDIM), jnp.float32)]
    scratch = [
        pltpu.VMEM((HALF, K), jnp.int32),
        pltpu.VMEM((HALF, K), jnp.int32),
        pltpu.VMEM((K, FDIM), jnp.float32),
        pltpu.VMEM((K, FDIM), jnp.float32),
        pltpu.VMEM((K,), jnp.float32),
        pltpu.VMEM_SHARED((NPAD, FDIM), jnp.float32),
        pltpu.VMEM_SHARED((NPAD,), jnp.float32),
        pltpu.SemaphoreType.DMA,
        pltpu.SemaphoreType.DMA,
    ]
    if with_deg:
        out_type.append(jax.ShapeDtypeStruct((NC * NPAD,), jnp.float32))

        def body(table, ei, z2, z1, agg_out, deg_out, *rest):
            _sc_agg_body(True, table, ei, z2, z1, agg_out, deg_out, *rest)
    else:
        def body(table, ei, z2, z1, agg_out, *rest):
            _sc_agg_body(False, table, ei, z2, z1, agg_out, None, *rest)

    f = pl.kernel(body, out_type=tuple(out_type), mesh=mesh,
                  scratch_types=scratch)
    return f(table, ei2d, z2, z1)


def _mm_relu_body(a0, a1, d0, d1, w, b, o):
    d = jnp.maximum(d0[...] + d1[...], 1.0)
    h = (a0[...] + a1[...]) / d
    y = jnp.dot(h, w[...], preferred_element_type=jnp.float32) + b[...]
    o[...] = jnp.maximum(y, 0.0)


def _mm_fused_body(a0, a1, d0, d1, w2, b2, w3, b3, o):
    d = jnp.maximum(d0[...] + d1[...], 1.0)
    h = (a0[...] + a1[...]) / d
    h = jnp.maximum(jnp.dot(h, w2[...], preferred_element_type=jnp.float32) + b2[...], 0.0)
    o[...] = jnp.dot(h, w3[...], preferred_element_type=jnp.float32) + b3[...]


_A0 = pl.BlockSpec((R, FDIM), lambda i: (i, 0))
_A1 = pl.BlockSpec((R, FDIM), lambda i: (i + GRID, 0))
_D0 = pl.BlockSpec((R, 1), lambda i: (i, 0))
_D1 = pl.BlockSpec((R, 1), lambda i: (i + GRID, 0))
_W = pl.BlockSpec((FDIM, FDIM), lambda i: (0, 0))
_B = pl.BlockSpec((1, FDIM), lambda i: (0, 0))
_O = pl.BlockSpec((R, FDIM), lambda i: (i, 0))


def _mm_relu(agg, deg2, w, b):
    return pl.pallas_call(
        _mm_relu_body,
        grid=(GRID,),
        in_specs=[_A0, _A1, _D0, _D1, _W, _B],
        out_specs=_O,
        out_shape=jax.ShapeDtypeStruct((NPAD, FDIM), jnp.float32),
    )(agg, agg, deg2, deg2, w, b)


def _mm_fused(agg, deg2, w2, b2, w3, b3):
    return pl.pallas_call(
        _mm_fused_body,
        grid=(GRID,),
        in_specs=[_A0, _A1, _D0, _D1, _W, _B, _W, _B],
        out_specs=_O,
        out_shape=jax.ShapeDtypeStruct((NPAD, FDIM), jnp.float32),
    )(agg, agg, deg2, deg2, w2, b2, w3, b3)


def kernel(x, edge_index, W1, b1, W2, b2, W3, b3):
    npd = E_PAD - N_EDGES
    pad_src = jnp.arange(npd, dtype=jnp.int32) % K
    pad_dst = N_NODES + jnp.arange(npd, dtype=jnp.int32) % (NPAD - N_NODES)
    src = jnp.concatenate([edge_index[0].astype(jnp.int32), pad_src])
    dst = jnp.concatenate([edge_index[1].astype(jnp.int32), pad_dst])
    ei2d = jnp.concatenate([src, dst]).reshape(2 * EROWS, K)
    z2 = jnp.zeros((NPAD, FDIM), jnp.float32)
    z1 = jnp.zeros((NPAD,), jnp.float32)

    agg1, deg = _sc_aggregate(x, ei2d, z2, z1, with_deg=True)
    deg2 = deg.reshape(NC * NPAD, 1)
    h1 = _mm_relu(agg1, deg2, W1, b1[None, :])
    (agg2,) = _sc_aggregate(h1, ei2d, z2, z1, with_deg=False)
    out = _mm_fused(agg2, deg2, W2, b2[None, :], W3, b3[None, :])
    return out[:N_NODES]


# speedup vs baseline: 3.2366x; 1.0881x over previous
import jax
import jax.numpy as jnp
from jax import lax
from jax.experimental import pallas as pl
from jax.experimental.pallas import tpu as pltpu
from jax.experimental.pallas import tpu_sc as plsc

N_NODES = 10000
N_EDGES = 320000
FDIM = 128

NC = 2
NS = 16
NW = NC * NS
K = 80
CHUNKS = 125
PAIRS = (CHUNKS - 1) // 2 - 1
NPAD = 10240
SLAB = NPAD // NS
EROWS = NW * CHUNKS

R = 512
GRID = NPAD // R


def _sc_agg_body(with_deg, table, ei, z2, z1, agg_out, deg_out,
                 s_idx, d_idx, rows0, rows1, ones, acc, deg_acc, sem0, sem1):
    c = lax.axis_index("c")
    s = lax.axis_index("s")
    slab = s * SLAB
    w = c * NS + s
    pltpu.sync_copy(z2, acc.at[pl.ds(slab, SLAB)])
    if with_deg:
        pltpu.sync_copy(z1, deg_acc.at[pl.ds(slab, SLAB)])
        for i in range(K // 16):
            ones[pl.ds(i * 16, 16)] = jnp.full((16,), 1.0, jnp.float32)
    pltpu.sync_copy(ei.at[pl.ds(w * CHUNKS, CHUNKS)], s_idx)
    pltpu.sync_copy(ei.at[pl.ds(EROWS + w * CHUNKS, CHUNKS)], d_idx)
    plsc.subcore_barrier()

    def gather_start(j, rows, sem):
        pltpu.async_copy(table.at[s_idx.at[j]], rows, sem)

    def gather_wait(j, rows, sem):
        pltpu.make_async_copy(table.at[s_idx.at[j]], rows, sem).wait()

    def scatter(j, rows):
        pltpu.sync_copy(rows, acc.at[d_idx.at[j]], add=True)
        if with_deg:
            pltpu.sync_copy(ones, deg_acc.at[d_idx.at[j]], add=True)

    gather_start(0, rows0, sem0)
    gather_start(1, rows1, sem1)

    def pair(t, carry):
        j = 2 * t
        gather_wait(j, rows0, sem0)
        scatter(j, rows0)
        gather_start(j + 2, rows0, sem0)
        gather_wait(j + 1, rows1, sem1)
        scatter(j + 1, rows1)
        gather_start(j + 3, rows1, sem1)
        return carry

    lax.fori_loop(0, PAIRS, pair, 0)
    gather_wait(CHUNKS - 3, rows0, sem0)
    scatter(CHUNKS - 3, rows0)
    gather_wait(CHUNKS - 2, rows1, sem1)
    scatter(CHUNKS - 2, rows1)
    gather_start(CHUNKS - 1, rows0, sem0)
    gather_wait(CHUNKS - 1, rows0, sem0)
    scatter(CHUNKS - 1, rows0)

    plsc.subcore_barrier()
    out_off = c * NPAD + slab
    pltpu.sync_copy(acc.at[pl.ds(slab, SLAB)], agg_out.at[pl.ds(out_off, SLAB)])
    if with_deg:
        pltpu.sync_copy(deg_acc.at[pl.ds(slab, SLAB)],
                        deg_out.at[pl.ds(out_off, SLAB)])


def _sc_aggregate(table, ei2d, z2, z1, with_deg):
    mesh = plsc.VectorSubcoreMesh(core_axis_name="c", subcore_axis_name="s")
    out_type = [jax.ShapeDtypeStruct((NC * NPAD, FDIM), jnp.float32)]
    scratch = [
        pltpu.VMEM((CHUNKS, K), jnp.int32),
        pltpu.VMEM((CHUNKS, K), jnp.int32),
        pltpu.VMEM((K, FDIM), jnp.float32),
        pltpu.VMEM((K, FDIM), jnp.float32),
        pltpu.VMEM((K,), jnp.float32),
        pltpu.VMEM_SHARED((NPAD, FDIM), jnp.float32),
        pltpu.VMEM_SHARED((NPAD,), jnp.float32),
        pltpu.SemaphoreType.DMA,
        pltpu.SemaphoreType.DMA,
    ]
    if with_deg:
        out_type.append(jax.ShapeDtypeStruct((NC * NPAD,), jnp.float32))

        def body(table, ei, z2, z1, agg_out, deg_out, *rest):
            _sc_agg_body(True, table, ei, z2, z1, agg_out, deg_out, *rest)
    else:
        def body(table, ei, z2, z1, agg_out, *rest):
            _sc_agg_body(False, table, ei, z2, z1, agg_out, None, *rest)

    f = pl.kernel(body, out_type=tuple(out_type), mesh=mesh,
                  scratch_types=scratch,
                  compiler_params=pltpu.CompilerParams(
                      use_tc_tiling_on_sc=False))
    return f(table, ei2d, z2, z1)


def _mm_relu_body(a0, a1, d0, d1, w, b, o):
    d = jnp.maximum(d0[...] + d1[...], 1.0)
    h = (a0[...] + a1[...]) / d
    y = jnp.dot(h, w[...], preferred_element_type=jnp.float32) + b[...]
    o[...] = jnp.maximum(y, 0.0)


def _mm_fused_body(a0, a1, d0, d1, w2, b2, w3, b3, o):
    d = jnp.maximum(d0[...] + d1[...], 1.0)
    h = (a0[...] + a1[...]) / d
    h = jnp.maximum(jnp.dot(h, w2[...], preferred_element_type=jnp.float32) + b2[...], 0.0)
    o[...] = jnp.dot(h, w3[...], preferred_element_type=jnp.float32) + b3[...]


_A0 = pl.BlockSpec((R, FDIM), lambda i: (i, 0))
_A1 = pl.BlockSpec((R, FDIM), lambda i: (i + GRID, 0))
_D0 = pl.BlockSpec((R, 1), lambda i: (i, 0))
_D1 = pl.BlockSpec((R, 1), lambda i: (i + GRID, 0))
_W = pl.BlockSpec((FDIM, FDIM), lambda i: (0, 0))
_B = pl.BlockSpec((1, FDIM), lambda i: (0, 0))
_O = pl.BlockSpec((R, FDIM), lambda i: (i, 0))


def _mm_relu(agg, deg2, w, b):
    return pl.pallas_call(
        _mm_relu_body,
        grid=(GRID,),
        in_specs=[_A0, _A1, _D0, _D1, _W, _B],
        out_specs=_O,
        out_shape=jax.ShapeDtypeStruct((NPAD, FDIM), jnp.float32),
    )(agg, agg, deg2, deg2, w, b)


def _mm_fused(agg, deg2, w2, b2, w3, b3):
    return pl.pallas_call(
        _mm_fused_body,
        grid=(GRID,),
        in_specs=[_A0, _A1, _D0, _D1, _W, _B, _W, _B],
        out_specs=_O,
        out_shape=jax.ShapeDtypeStruct((NPAD, FDIM), jnp.float32),
    )(agg, agg, deg2, deg2, w2, b2, w3, b3)


def kernel(x, edge_index, W1, b1, W2, b2, W3, b3):
    ei2d = edge_index.astype(jnp.int32).reshape(2 * EROWS, K)
    z2 = jnp.zeros((SLAB, FDIM), jnp.float32)
    z1 = jnp.zeros((SLAB,), jnp.float32)

    agg1, deg = _sc_aggregate(x, ei2d, z2, z1, with_deg=True)
    deg2 = deg.reshape(NC * NPAD, 1)
    h1 = _mm_relu(agg1, deg2, W1, b1[None, :])
    (agg2,) = _sc_aggregate(h1, ei2d, z2, z1, with_deg=False)
    out = _mm_fused(agg2, deg2, W2, b2[None, :], W3, b3[None, :])
    return out[:N_NODES]
